# Initial kernel scaffold; baseline (speedup 1.0000x reference)
import jax
import jax.numpy as jnp
from jax.experimental import pallas as pl


def kernel(problem_reps, centroids, kmeans_log_tau):
    raise NotImplementedError("write your pallas kernel here")


# speedup vs baseline: 1.9760x; 1.9760x over previous
import functools

import jax
import jax.numpy as jnp
from jax.experimental import pallas as pl
from jax.experimental.pallas import tpu as pltpu


def _kc_pass_kernel(tau_ref, x_ref, c_ref, *refs, nb, emit_logits):
    if emit_logits:
        logits_ref, c_out_ref, acc_ref, w_ref = refs
    else:
        c_out_ref, acc_ref, w_ref = refs
        logits_ref = None

    j = pl.program_id(0)

    @pl.when(j == 0)
    def _init():
        acc_ref[...] = jnp.zeros_like(acc_ref)
        w_ref[...] = jnp.zeros_like(w_ref)

    x = x_ref[...]
    c = c_ref[...]
    tau = tau_ref[0]

    x2 = jnp.sum(x * x, axis=1, keepdims=True)
    c2 = jnp.sum(c * c, axis=1)[None, :]
    xc = jax.lax.dot_general(
        x, c, (((1,), (1,)), ((), ())),
        preferred_element_type=jnp.float32,
    )
    dist = x2 - 2.0 * xc + c2
    logits = -dist / tau
    if emit_logits:
        logits_ref[...] = logits

    m = jnp.max(logits, axis=1, keepdims=True)
    e = jnp.exp(logits - m)
    s = jnp.sum(e, axis=1, keepdims=True)
    assign = e / s

    w_ref[...] += jnp.sum(assign, axis=0, keepdims=True)
    acc_ref[...] += jax.lax.dot_general(
        assign, x, (((0,), (0,)), ((), ())),
        preferred_element_type=jnp.float32,
    )

    @pl.when(j == nb - 1)
    def _finish():
        w_col = jnp.transpose(w_ref[...])
        c_out_ref[...] = acc_ref[...] / (w_col + 1e-8)


def _run_pass(tau, x, c, *, block_n, emit_logits, interpret=False):
    n, d = x.shape
    k = c.shape[0]
    nb = n // block_n
    scratch = [
        pltpu.VMEM((k, d), jnp.float32),
        pltpu.VMEM((1, k), jnp.float32),
    ]
    in_specs = [
        pl.BlockSpec(memory_space=pltpu.SMEM),
        pl.BlockSpec((block_n, d), lambda j: (j, 0)),
        pl.BlockSpec((k, d), lambda j: (0, 0)),
    ]
    c_spec = pl.BlockSpec((k, d), lambda j: (0, 0))
    c_shape = jax.ShapeDtypeStruct((k, d), jnp.float32)
    if emit_logits:
        out_specs = [pl.BlockSpec((block_n, k), lambda j: (j, 0)), c_spec]
        out_shape = [jax.ShapeDtypeStruct((n, k), jnp.float32), c_shape]
    else:
        out_specs = c_spec
        out_shape = c_shape
    return pl.pallas_call(
        functools.partial(_kc_pass_kernel, nb=nb, emit_logits=emit_logits),
        grid=(nb,),
        in_specs=in_specs,
        out_specs=out_specs,
        out_shape=out_shape,
        scratch_shapes=scratch,
        interpret=interpret,
    )(tau, x, c)


def kernel(problem_reps, centroids, kmeans_log_tau):
    tau = jnp.exp(kmeans_log_tau)
    block_n = 512
    c1 = _run_pass(tau, problem_reps, centroids,
                   block_n=block_n, emit_logits=False)
    logits, c2 = _run_pass(tau, problem_reps, c1,
                           block_n=block_n, emit_logits=True)
    return logits, c2


# speedup vs baseline: 2.6574x; 1.3449x over previous
import functools

import jax
import jax.numpy as jnp
from jax.experimental import pallas as pl
from jax.experimental.pallas import tpu as pltpu


def _kc_pass_kernel(scal_ref, x_ref, xt_ref, ct_ref, *refs, nb, emit_logits):
    if emit_logits:
        logits_ref, cout_t_ref, acc_ref, w_ref, b_ref = refs
    else:
        cout_t_ref, acc_ref, w_ref, b_ref = refs
        logits_ref = None

    j = pl.program_id(0)
    inv_tau = scal_ref[0]
    ct = ct_ref[...]

    @pl.when(j == 0)
    def _init():
        acc_ref[...] = jnp.zeros_like(acc_ref)
        w_ref[...] = jnp.zeros_like(w_ref)
        b_ref[...] = jnp.sum(ct * ct, axis=0, keepdims=True) * inv_tau

    x = x_ref[...]
    xs = x * (2.0 * inv_tau)
    xc = jnp.dot(xs, ct, preferred_element_type=jnp.float32)
    g = xc - b_ref[...]

    m = jnp.max(g, axis=1, keepdims=True)
    e = jnp.exp(g - m)
    s = jnp.sum(e, axis=1, keepdims=True)
    assign = e / s

    if emit_logits:
        x2 = jnp.sum(x * x, axis=1, keepdims=True) * inv_tau
        logits_ref[...] = g - x2

    w_ref[...] += jnp.sum(assign, axis=0, keepdims=True)
    acc_ref[...] += jnp.dot(xt_ref[...], assign,
                            preferred_element_type=jnp.float32)

    @pl.when(j == nb - 1)
    def _finish():
        cout_t_ref[...] = acc_ref[...] / (w_ref[...] + 1e-8)


def _run_pass(inv_tau, x, xt, ct, *, block_n, emit_logits, interpret=False):
    n, d = x.shape
    k = ct.shape[1]
    nb = n // block_n
    scratch = [
        pltpu.VMEM((d, k), jnp.float32),
        pltpu.VMEM((1, k), jnp.float32),
        pltpu.VMEM((1, k), jnp.float32),
    ]
    in_specs = [
        pl.BlockSpec(memory_space=pltpu.SMEM),
        pl.BlockSpec((block_n, d), lambda j: (j, 0)),
        pl.BlockSpec((d, block_n), lambda j: (0, j)),
        pl.BlockSpec((d, k), lambda j: (0, 0)),
    ]
    ct_spec = pl.BlockSpec((d, k), lambda j: (0, 0))
    ct_shape = jax.ShapeDtypeStruct((d, k), jnp.float32)
    if emit_logits:
        out_specs = [pl.BlockSpec((block_n, k), lambda j: (j, 0)), ct_spec]
        out_shape = [jax.ShapeDtypeStruct((n, k), jnp.float32), ct_shape]
    else:
        out_specs = ct_spec
        out_shape = ct_shape
    return pl.pallas_call(
        functools.partial(_kc_pass_kernel, nb=nb, emit_logits=emit_logits),
        grid=(nb,),
        in_specs=in_specs,
        out_specs=out_specs,
        out_shape=out_shape,
        scratch_shapes=scratch,
        interpret=interpret,
    )(inv_tau, x, xt, ct)


def kernel(problem_reps, centroids, kmeans_log_tau):
    inv_tau = 1.0 / jnp.exp(kmeans_log_tau)
    x = problem_reps
    xt = jnp.transpose(x)
    ct0 = jnp.transpose(centroids)
    block_n = 512
    c1t = _run_pass(inv_tau, x, xt, ct0,
                    block_n=block_n, emit_logits=False)
    logits, c2t = _run_pass(inv_tau, x, xt, c1t,
                            block_n=block_n, emit_logits=True)
    return logits, jnp.transpose(c2t)
